# baseline (speedup 1.0000x reference)
import functools

import jax
import jax.numpy as jnp
from jax import lax
from jax.experimental import pallas as pl
from jax.experimental.pallas import tpu as pltpu
from jax.experimental.pallas import tpu_sc as plsc

B = 4096
D = 64
H1 = 32
HIST = 50
VOCAB = 100000
NC = 2
NS = 16
NW = NC * NS
BPW = B // NW
RPC = 2
NCHUNK = BPW // RPC
IDX_PER_CHUNK = RPC * HIST
NBUF = 4
LN = 16

GRP = 8
VB = 16384
SB = VB // GRP
NBLK = (VOCAB + VB - 1) // VB
VOCABP = NBLK * VB

_mesh = plsc.VectorSubcoreMesh(
    core_axis_name="c", subcore_axis_name="s", num_cores=NC, num_subcores=NS
)


def _project_body(tt_ref, w1_ref, out_ref):
    h = lax.dot_general(
        tt_ref[...].astype(jnp.bfloat16), w1_ref[...].astype(jnp.bfloat16),
        (((0,), (0,)), ((), ())),
        preferred_element_type=jnp.float32,
    )
    h = h.astype(jnp.bfloat16)
    ws = []
    for k in range(GRP):
        hk = h[k * SB:(k + 1) * SB, :]
        eb = lax.bitcast_convert_type(hk[:, :H1 // 2], jnp.uint16).astype(jnp.int32)
        ob = lax.bitcast_convert_type(hk[:, H1 // 2:], jnp.uint16).astype(jnp.int32)
        ws.append(eb | (ob << 16))
    out_ref[...] = jnp.concatenate(ws, axis=1)


_project = pl.pallas_call(
    _project_body,
    grid=(NBLK,),
    in_specs=[
        pl.BlockSpec((D, VB), lambda j: (0, j)),
        pl.BlockSpec((D, H1), lambda j: (0, 0)),
    ],
    out_specs=pl.BlockSpec((SB, 128), lambda j: (j, 0)),
    out_shape=jax.ShapeDtypeStruct((NBLK * SB, 128), jnp.int32),
)


@functools.partial(
    pl.kernel,
    out_type=jax.ShapeDtypeStruct((B,), jnp.float32),
    mesh=_mesh,
    scratch_types=[
        pltpu.VMEM((NCHUNK, IDX_PER_CHUNK), jnp.int32),
        pltpu.VMEM((NCHUNK, 2 * 64), jnp.int32),
        pltpu.VMEM((64,), jnp.float32),
        [pltpu.VMEM((IDX_PER_CHUNK, H1 // 2), jnp.int32) for _ in range(NBUF)],
        pltpu.VMEM((BPW,), jnp.float32),
        [pltpu.SemaphoreType.DMA for _ in range(NBUF)],
    ],
    compiler_params=pltpu.CompilerParams(use_tc_tiling_on_sc=False, needs_layout_passes=False),
)
def _pool(xp_hbm, xq_hbm, bw_hbm, t1_hbm, out_hbm, idx_v, cnt_v, bw_v, bufs,
          out_v, sems):
    wid = lax.axis_index("s") * NC + lax.axis_index("c")
    pltpu.sync_copy(xp_hbm.at[pl.ds(wid * NCHUNK, NCHUNK)], idx_v)
    pltpu.sync_copy(xq_hbm.at[pl.ds(wid * NCHUNK, NCHUNK)], cnt_v)
    pltpu.sync_copy(bw_hbm, bw_v)
    for b in range(NBUF):
        pltpu.async_copy(t1_hbm.at[idx_v.at[b]], bufs[b], sems[b])

    b1e = bw_v[pl.ds(0, LN)]
    b1o = bw_v[pl.ds(LN, LN)]
    w2e = bw_v[pl.ds(2 * LN, LN)]
    w2o = bw_v[pl.ds(3 * LN, LN)]

    def _process(j, buf):
        acc = [
            [jnp.zeros((LN,), jnp.float32) for _ in range(2)]
            for _ in range(RPC)
        ]
        for r in range(HIST):
            for half in range(RPC):
                w = buf[half * HIST + r, :]
                ev = plsc.bitcast(w << 16, jnp.float32)
                od = plsc.bitcast(w & jnp.int32(-65536), jnp.float32)
                acc[half][0] = acc[half][0] + ev
                acc[half][1] = acc[half][1] + od
        for half in range(RPC):
            m = jnp.zeros((LN,), jnp.float32)
            for k in range(4):
                iv = cnt_v[j, pl.ds(half * 64 + k * LN, LN)]
                m = m + jnp.where(iv != 0, 1.0, 0.0)
            tot = jnp.sum(m, axis=0)
            cvec = jnp.maximum(jnp.full((LN,), tot, jnp.float32), 1.0)
            he = jnp.maximum(acc[half][0] / cvec + b1e, 0.0)
            ho = jnp.maximum(acc[half][1] / cvec + b1o, 0.0)
            o = jnp.full((LN,), jnp.sum(he * w2e + ho * w2o, axis=0))
            g = j * RPC + half
            base = pl.multiple_of((g >> 4) << 4, LN)
            lane = g & (LN - 1)
            cur = out_v[pl.ds(base, LN)]
            out_v[pl.ds(base, LN)] = jnp.where(
                lax.iota(jnp.int32, LN) == lane, o, cur
            )

    def _tbody(t, carry):
        for b in range(NBUF):
            j = NBUF * t + b
            pltpu.make_async_copy(t1_hbm.at[idx_v.at[j]], bufs[b], sems[b]).wait()
            _process(j, bufs[b])

            @pl.when(j + NBUF < NCHUNK)
            def _():
                pltpu.async_copy(t1_hbm.at[idx_v.at[j + NBUF]], bufs[b], sems[b])

        return carry

    lax.fori_loop(0, NCHUNK // NBUF, _tbody, 0)
    pltpu.sync_copy(out_v, out_hbm.at[pl.ds(wid * BPW, BPW)])


_PERM = tuple(range(0, H1, 2)) + tuple(range(1, H1, 2))


def kernel(x, table, W1, b1, W2, b2):
    x = x.astype(jnp.int32)
    w1eo = jnp.concatenate([W1[:, 0::2], W1[:, 1::2]], axis=1)
    t1g = _project(table.T, w1eo)
    t1 = t1g.reshape(VOCABP, H1 // 2)
    xt = (x & ~(VB - 1)) + ((x & (SB - 1)) << 3) + ((x & (VB - 1)) >> (SB.bit_length() - 1))
    xp = xt.reshape(B // RPC, RPC * HIST)
    xq = jnp.pad(x, ((0, 0), (0, 64 - HIST))).reshape(B // RPC, RPC * 64)
    perm = jnp.asarray(_PERM, dtype=jnp.int32)
    bw = jnp.concatenate([b1[perm], W2[perm, 0]])
    out = _pool(xp, xq, bw, t1)
    return out.reshape(B, 1) + b2


# speedup vs baseline: 1.0292x; 1.0292x over previous
import functools

import jax
import jax.numpy as jnp
from jax import lax
from jax.experimental import pallas as pl
from jax.experimental.pallas import tpu as pltpu
from jax.experimental.pallas import tpu_sc as plsc

B = 4096
D = 64
H1 = 32
HIST = 50
VOCAB = 100000
NC = 2
NS = 16
NW = NC * NS
BPW = B // NW
RPC = 2
NCHUNK = BPW // RPC
IDX_PER_CHUNK = RPC * HIST
NBUF = 4
LN = 16

GRP = 8
VB = 8192
SB = VB // GRP
NBLK = (VOCAB + VB - 1) // VB
VOCABP = NBLK * VB

_mesh = plsc.VectorSubcoreMesh(
    core_axis_name="c", subcore_axis_name="s", num_cores=NC, num_subcores=NS
)


def _project_body(tt_ref, w1_ref, out_ref):
    h = lax.dot_general(
        tt_ref[...].astype(jnp.bfloat16), w1_ref[...].astype(jnp.bfloat16),
        (((0,), (0,)), ((), ())),
        preferred_element_type=jnp.float32,
    )
    h = h.astype(jnp.bfloat16)
    ws = []
    for k in range(GRP):
        hk = h[k * SB:(k + 1) * SB, :]
        eb = lax.bitcast_convert_type(hk[:, :H1 // 2], jnp.uint16).astype(jnp.int32)
        ob = lax.bitcast_convert_type(hk[:, H1 // 2:], jnp.uint16).astype(jnp.int32)
        ws.append(eb | (ob << 16))
    out_ref[...] = jnp.concatenate(ws, axis=1)


_project = pl.pallas_call(
    _project_body,
    grid=(NBLK,),
    in_specs=[
        pl.BlockSpec((D, VB), lambda j: (0, j)),
        pl.BlockSpec((D, H1), lambda j: (0, 0)),
    ],
    out_specs=pl.BlockSpec((SB, 128), lambda j: (j, 0)),
    out_shape=jax.ShapeDtypeStruct((NBLK * SB, 128), jnp.int32),
)


@functools.partial(
    pl.kernel,
    out_type=jax.ShapeDtypeStruct((B,), jnp.float32),
    mesh=_mesh,
    scratch_types=[
        pltpu.VMEM((NCHUNK, IDX_PER_CHUNK), jnp.int32),
        pltpu.VMEM((NCHUNK, 2 * 64), jnp.int32),
        pltpu.VMEM((64,), jnp.float32),
        [pltpu.VMEM((IDX_PER_CHUNK, H1 // 2), jnp.int32) for _ in range(NBUF)],
        pltpu.VMEM((BPW,), jnp.float32),
        [pltpu.SemaphoreType.DMA for _ in range(NBUF)],
    ],
    compiler_params=pltpu.CompilerParams(use_tc_tiling_on_sc=False, needs_layout_passes=False),
)
def _pool(xp_hbm, xq_hbm, bw_hbm, t1_hbm, out_hbm, idx_v, cnt_v, bw_v, bufs,
          out_v, sems):
    wid = lax.axis_index("s") * NC + lax.axis_index("c")
    pltpu.sync_copy(xp_hbm.at[pl.ds(wid * NCHUNK, NCHUNK)], idx_v)
    pltpu.sync_copy(xq_hbm.at[pl.ds(wid * NCHUNK, NCHUNK)], cnt_v)
    pltpu.sync_copy(bw_hbm, bw_v)
    for b in range(NBUF):
        pltpu.async_copy(t1_hbm.at[idx_v.at[b]], bufs[b], sems[b])

    b1e = bw_v[pl.ds(0, LN)]
    b1o = bw_v[pl.ds(LN, LN)]
    w2e = bw_v[pl.ds(2 * LN, LN)]
    w2o = bw_v[pl.ds(3 * LN, LN)]

    def _process(j, buf):
        acc = [
            [jnp.zeros((LN,), jnp.float32) for _ in range(2)]
            for _ in range(RPC)
        ]
        for r in range(HIST):
            for half in range(RPC):
                w = buf[half * HIST + r, :]
                ev = plsc.bitcast(w << 16, jnp.float32)
                od = plsc.bitcast(w & jnp.int32(-65536), jnp.float32)
                acc[half][0] = acc[half][0] + ev
                acc[half][1] = acc[half][1] + od
        for half in range(RPC):
            m = jnp.zeros((LN,), jnp.float32)
            for k in range(4):
                iv = cnt_v[j, pl.ds(half * 64 + k * LN, LN)]
                m = m + jnp.where(iv != 0, 1.0, 0.0)
            tot = jnp.sum(m, axis=0)
            cvec = jnp.maximum(jnp.full((LN,), tot, jnp.float32), 1.0)
            he = jnp.maximum(acc[half][0] / cvec + b1e, 0.0)
            ho = jnp.maximum(acc[half][1] / cvec + b1o, 0.0)
            o = jnp.full((LN,), jnp.sum(he * w2e + ho * w2o, axis=0))
            g = j * RPC + half
            base = pl.multiple_of((g >> 4) << 4, LN)
            lane = g & (LN - 1)
            cur = out_v[pl.ds(base, LN)]
            out_v[pl.ds(base, LN)] = jnp.where(
                lax.iota(jnp.int32, LN) == lane, o, cur
            )

    def _tbody(t, carry):
        for b in range(NBUF):
            j = NBUF * t + b
            pltpu.make_async_copy(t1_hbm.at[idx_v.at[j]], bufs[b], sems[b]).wait()
            _process(j, bufs[b])

            @pl.when(j + NBUF < NCHUNK)
            def _():
                pltpu.async_copy(t1_hbm.at[idx_v.at[j + NBUF]], bufs[b], sems[b])

        return carry

    lax.fori_loop(0, NCHUNK // NBUF, _tbody, 0)
    pltpu.sync_copy(out_v, out_hbm.at[pl.ds(wid * BPW, BPW)])


_PERM = tuple(range(0, H1, 2)) + tuple(range(1, H1, 2))


def kernel(x, table, W1, b1, W2, b2):
    x = x.astype(jnp.int32)
    w1eo = jnp.concatenate([W1[:, 0::2], W1[:, 1::2]], axis=1)
    t1g = _project(table.T, w1eo)
    t1 = t1g.reshape(VOCABP, H1 // 2)
    xt = (x & ~(VB - 1)) + ((x & (SB - 1)) << 3) + ((x & (VB - 1)) >> (SB.bit_length() - 1))
    xp = xt.reshape(B // RPC, RPC * HIST)
    xq = jnp.pad(x, ((0, 0), (0, 64 - HIST))).reshape(B // RPC, RPC * 64)
    perm = jnp.asarray(_PERM, dtype=jnp.int32)
    bw = jnp.concatenate([b1[perm], W2[perm, 0]])
    out = _pool(xp, xq, bw, t1)
    return out.reshape(B, 1) + b2
